# baseline (speedup 1.0000x reference)
import jax
import jax.numpy as jnp
from jax import lax
from jax.experimental import pallas as pl
from jax.experimental.pallas import tpu as pltpu
from jax.experimental.pallas import tpu_sc as plsc

N_VOX = 500_000
NW = 32
CH = 1024
IN_CH = 4 * CH
OUT_CH = 32 * CH
FULL_CHUNKS = N_VOX // CH
TAIL_VOX = N_VOX - FULL_CHUNKS * CH
BASE_ITERS = FULL_CHUNKS // NW
EXTRA_W = FULL_CHUNKS % NW


def _tec_body(x_hbm, out_hbm, in_v, out_v):
    nc = 2
    w = lax.axis_index("s") * nc + lax.axis_index("c")

    lane = lax.iota(jnp.int32, 16)
    j = lane % 4
    mulv = jnp.where(j == 3, 1, 2).astype(jnp.int32)
    off0 = (16912 >> lane) & 1
    off1 = (30051 >> lane) & 1

    def compute(nvox):
        def body(v, carry):
            g = plsc.load_gather(in_v, [j + 4 * v])
            gm = g * mulv
            out_v[pl.ds(32 * v, 16)] = gm + off0
            out_v[pl.ds(32 * v + 16, 16)] = gm + off1
            return carry
        lax.fori_loop(0, nvox, body, 0)

    def do_chunk(cid):
        pltpu.sync_copy(x_hbm.at[pl.ds(cid * IN_CH, IN_CH)], in_v)
        compute(CH)
        pltpu.sync_copy(out_v, out_hbm.at[pl.ds(cid * OUT_CH, OUT_CH)])

    def loop_body(i, carry):
        do_chunk(w + NW * i)
        return carry
    lax.fori_loop(0, BASE_ITERS, loop_body, 0)

    @pl.when(w < EXTRA_W)
    def _():
        do_chunk(BASE_ITERS * NW + w)

    @pl.when(w == EXTRA_W)
    def _():
        t_in = 4 * TAIL_VOX
        t_out = 32 * TAIL_VOX
        pltpu.sync_copy(x_hbm.at[pl.ds(FULL_CHUNKS * IN_CH, t_in)],
                        in_v.at[pl.ds(0, t_in)])
        compute(TAIL_VOX)
        pltpu.sync_copy(out_v.at[pl.ds(0, t_out)],
                        out_hbm.at[pl.ds(FULL_CHUNKS * OUT_CH, t_out)])


def kernel(voxel_inds):
    x = voxel_inds.reshape(-1)
    mesh = plsc.VectorSubcoreMesh(core_axis_name="c", subcore_axis_name="s")
    out = pl.kernel(
        _tec_body,
        out_type=jax.ShapeDtypeStruct((N_VOX * 32,), jnp.int32),
        mesh=mesh,
        compiler_params=pltpu.CompilerParams(needs_layout_passes=False),
        scratch_types=[
            pltpu.VMEM((IN_CH,), jnp.int32),
            pltpu.VMEM((OUT_CH,), jnp.int32),
        ],
    )(x)
    return out.reshape(-1, 4)


# speedup vs baseline: 3.4932x; 3.4932x over previous
import jax
import jax.numpy as jnp
from jax import lax
from jax.experimental import pallas as pl
from jax.experimental.pallas import tpu as pltpu
from jax.experimental.pallas import tpu_sc as plsc

N_VOX = 500_000
N_OUT_TILES = N_VOX * 8 // 128
NW = 32
CH = 1024
TPC = CH // 16
FULL_CHUNKS = N_VOX // CH
TAIL_VOX = N_VOX - FULL_CHUNKS * CH
TAIL_TILES = TAIL_VOX // 16
BASE_ITERS = FULL_CHUNKS // NW
EXTRA_W = FULL_CHUNKS % NW

_OFF_BITS = (210, 180, 232, 0)
_MULS = (2, 2, 2, 1)


def _tec_body(x_hbm, out_hbm, in_v, out_v):
    nc = 2
    w = lax.axis_index("s") * nc + lax.axis_index("c")

    lane = lax.iota(jnp.int32, 16)
    duo = lane // 8
    corner = lane % 8
    offv = [((b >> corner) & 1).astype(jnp.int32) for b in _OFF_BITS]
    jvec = [jnp.broadcast_to(jnp.int32(j), (16,)) for j in range(4)]

    def compute(ntiles):
        def body(g, carry):
            vox0 = 16 * g
            for s in range(8):
                rows = vox0 + 2 * s + duo
                for j in range(4):
                    gj = plsc.load_gather(in_v, [rows, jvec[j]])
                    out_v[g, j, pl.ds(16 * s, 16)] = gj * _MULS[j] + offv[j]
            return carry
        lax.fori_loop(0, ntiles, body, 0)

    def do_chunk(cid, nvox, ntiles):
        pltpu.sync_copy(x_hbm.at[pl.ds(cid * CH, nvox)],
                        in_v.at[pl.ds(0, nvox)])
        compute(ntiles)
        pltpu.sync_copy(out_v.at[pl.ds(0, ntiles)],
                        out_hbm.at[pl.ds(cid * TPC, ntiles)])

    def loop_body(i, carry):
        do_chunk(w + NW * i, CH, TPC)
        return carry
    lax.fori_loop(0, BASE_ITERS, loop_body, 0)

    @pl.when(w < EXTRA_W)
    def _():
        do_chunk(BASE_ITERS * NW + w, CH, TPC)

    @pl.when(w == EXTRA_W)
    def _():
        do_chunk(FULL_CHUNKS, TAIL_VOX, TAIL_TILES)


def kernel(voxel_inds):
    mesh = plsc.VectorSubcoreMesh(core_axis_name="c", subcore_axis_name="s")
    out3 = pl.kernel(
        _tec_body,
        out_type=jax.ShapeDtypeStruct((N_OUT_TILES, 4, 128), jnp.int32),
        mesh=mesh,
        compiler_params=pltpu.CompilerParams(
            needs_layout_passes=False, use_tc_tiling_on_sc=False),
        scratch_types=[
            pltpu.VMEM((CH, 4), jnp.int32),
            pltpu.VMEM((TPC, 4, 128), jnp.int32),
        ],
    )(voxel_inds)
    return out3.transpose(0, 2, 1).reshape(-1, 4)


# speedup vs baseline: 16.3264x; 4.6737x over previous
import jax
import jax.numpy as jnp
from jax import lax
from jax.experimental import pallas as pl
from jax.experimental.pallas import tpu as pltpu
from jax.experimental.pallas import tpu_sc as plsc

N_VOX = 500_000
N_HEAD = N_VOX // 128 * 128
N_IN_TILES = N_HEAD // 128
N_TAIL = N_VOX - N_HEAD
N_OUT_TILES = N_VOX * 8 // 128
NW = 32
CH = 1024
IPC = CH // 128
TPC = CH // 16
FULL_CHUNKS = N_HEAD // CH
HEAD_REM_VOX = N_HEAD - FULL_CHUNKS * CH
BASE_ITERS = FULL_CHUNKS // NW
EXTRA_W = FULL_CHUNKS % NW

_OFF_BITS = (210, 180, 232, 0)
_MULS = (2, 2, 2, 1)


def _tec_body(xh_hbm, xt_hbm, out_hbm, in_v, in_t, out_v):
    nc = 2
    w = lax.axis_index("s") * nc + lax.axis_index("c")

    lane = lax.iota(jnp.int32, 16)
    duo = lane // 8
    corner = lane % 8
    offv = [((b >> corner) & 1).astype(jnp.int32) for b in _OFF_BITS]
    jvec = [jnp.broadcast_to(jnp.int32(j), (16,)) for j in range(4)]

    def compute(ntiles):
        def body(g, carry):
            vox0 = 16 * g
            for s in range(8):
                vox = vox0 + 2 * s + duo
                t_idx = vox >> 7
                l_idx = vox & 127
                for j in range(4):
                    gj = plsc.load_gather(in_v, [t_idx, jvec[j], l_idx])
                    out_v[g, j, pl.ds(16 * s, 16)] = gj * _MULS[j] + offv[j]
            return carry
        lax.fori_loop(0, ntiles, body, 0)

    def do_chunk(cid):
        pltpu.sync_copy(xh_hbm.at[pl.ds(cid * IPC, IPC)], in_v)
        compute(TPC)
        pltpu.sync_copy(out_v, out_hbm.at[pl.ds(cid * TPC, TPC)])

    def loop_body(i, carry):
        do_chunk(w + NW * i)
        return carry
    lax.fori_loop(0, BASE_ITERS, loop_body, 0)

    @pl.when(w < EXTRA_W)
    def _():
        do_chunk(BASE_ITERS * NW + w)

    @pl.when(w == EXTRA_W)
    def _():
        pltpu.sync_copy(xh_hbm.at[pl.ds(FULL_CHUNKS * IPC, 2)],
                        in_v.at[pl.ds(0, 2)])
        compute(16)
        pltpu.sync_copy(xt_hbm, in_t)
        for g in range(2):
            for s in range(8):
                rows = 16 * g + 2 * s + duo
                for j in range(4):
                    gj = plsc.load_gather(in_t, [rows, jvec[j]])
                    out_v[16 + g, j, pl.ds(16 * s, 16)] = (
                        gj * _MULS[j] + offv[j])
        pltpu.sync_copy(out_v.at[pl.ds(0, 18)],
                        out_hbm.at[pl.ds(FULL_CHUNKS * TPC, 18)])


def kernel(voxel_inds):
    xh = voxel_inds[:N_HEAD].reshape(N_IN_TILES, 128, 4).transpose(0, 2, 1)
    xt = voxel_inds[N_HEAD:]
    mesh = plsc.VectorSubcoreMesh(core_axis_name="c", subcore_axis_name="s")
    out3 = pl.kernel(
        _tec_body,
        out_type=jax.ShapeDtypeStruct((N_OUT_TILES, 4, 128), jnp.int32),
        mesh=mesh,
        compiler_params=pltpu.CompilerParams(
            needs_layout_passes=False, use_tc_tiling_on_sc=False),
        scratch_types=[
            pltpu.VMEM((IPC, 4, 128), jnp.int32),
            pltpu.VMEM((N_TAIL, 4), jnp.int32),
            pltpu.VMEM((TPC, 4, 128), jnp.int32),
        ],
    )(xh, xt)
    return out3.transpose(0, 2, 1).reshape(-1, 4)


# speedup vs baseline: 20.1217x; 1.2325x over previous
import jax
import jax.numpy as jnp
from jax import lax
from jax.experimental import pallas as pl
from jax.experimental.pallas import tpu as pltpu
from jax.experimental.pallas import tpu_sc as plsc

N_VOX = 500_000
N_HEAD = N_VOX // 128 * 128
N_IN_TILES = N_HEAD // 128
N_TAIL = N_VOX - N_HEAD
N_OUT_TILES = N_VOX * 8 // 128
NW = 32
CH = 1024
IPC = CH // 128
TPC = CH // 16
FULL_CHUNKS = N_HEAD // CH
HEAD_REM_VOX = N_HEAD - FULL_CHUNKS * CH
BASE_ITERS = FULL_CHUNKS // NW
EXTRA_W = FULL_CHUNKS % NW

_OFF_BITS = (210, 180, 232, 0)
_MULS = (2, 2, 2, 1)


def _tec_body(xh_hbm, xt_hbm, out_hbm, in_v, in_t, out_v, sem_in, sem_out):
    nc = 2
    w = lax.axis_index("s") * nc + lax.axis_index("c")

    lane = lax.iota(jnp.int32, 16)
    duo = lane // 8
    corner = lane % 8
    offv = [((b >> corner) & 1).astype(jnp.int32) for b in _OFF_BITS[:3]]
    jvec = [jnp.broadcast_to(jnp.int32(j), (16,)) for j in range(4)]

    nits = jnp.where(w < EXTRA_W, BASE_ITERS + 1, BASE_ITERS)

    def start_in(i, buf):
        cid = w + NW * i
        pltpu.make_async_copy(
            xh_hbm.at[pl.ds(cid * IPC, IPC)],
            in_v.at[pl.ds(buf * IPC, IPC)], sem_in).start()

    def wait_in():
        pltpu.make_async_copy(
            xh_hbm.at[pl.ds(0, IPC)], in_v.at[pl.ds(0, IPC)], sem_in).wait()

    def start_out(i, buf):
        cid = w + NW * i
        pltpu.make_async_copy(
            out_v.at[pl.ds(buf * TPC, TPC)],
            out_hbm.at[pl.ds(cid * TPC, TPC)], sem_out).start()

    def wait_out():
        pltpu.make_async_copy(
            out_v.at[pl.ds(0, TPC)],
            out_hbm.at[pl.ds(0, TPC)], sem_out).wait()

    def compute_chunk(buf):
        ib = buf * IPC
        ob = buf * TPC

        def dbl(bt, carry):
            t = ib + bt
            for j in range(3):
                for v in range(8):
                    sl = pl.ds(16 * v, 16)
                    in_v[t, j, sl] = in_v[t, j, sl] * 2
            return carry
        lax.fori_loop(0, IPC, dbl, 0)

        def blk(bt, carry):
            t = ib + bt
            tv = jnp.broadcast_to(t, (16,))
            og0 = ob + 8 * bt
            for gg in range(8):
                for s in range(8):
                    lv = duo + (16 * gg + 2 * s)
                    osl = pl.ds(16 * s, 16)
                    for j in range(3):
                        gj = plsc.load_gather(in_v, [tv, jvec[j], lv])
                        out_v[og0 + gg, j, osl] = gj + offv[j]
                    g3 = plsc.load_gather(in_v, [tv, jvec[3], lv])
                    out_v[og0 + gg, 3, osl] = g3
            return carry
        lax.fori_loop(0, IPC, blk, 0)

    start_in(0, 0)

    def loop_body(i, carry):
        buf = i & 1
        wait_in()

        @pl.when(i + 1 < nits)
        def _():
            start_in(i + 1, buf ^ 1)

        @pl.when(i >= 2)
        def _():
            wait_out()

        compute_chunk(buf)
        start_out(i, buf)
        return carry
    lax.fori_loop(0, nits, loop_body, 0)

    wait_out()
    wait_out()

    @pl.when(w == EXTRA_W)
    def _():
        pltpu.sync_copy(xh_hbm.at[pl.ds(FULL_CHUNKS * IPC, 2)],
                        in_v.at[pl.ds(0, 2)])
        for bt in range(2):
            for j in range(3):
                for v in range(8):
                    sl = pl.ds(16 * v, 16)
                    in_v[bt, j, sl] = in_v[bt, j, sl] * 2
            tv = jnp.broadcast_to(jnp.int32(bt), (16,))
            for gg in range(8):
                for s in range(8):
                    lv = duo + (16 * gg + 2 * s)
                    osl = pl.ds(16 * s, 16)
                    for j in range(3):
                        gj = plsc.load_gather(in_v, [tv, jvec[j], lv])
                        out_v[8 * bt + gg, j, osl] = gj + offv[j]
                    g3 = plsc.load_gather(in_v, [tv, jvec[3], lv])
                    out_v[8 * bt + gg, 3, osl] = g3
        pltpu.sync_copy(xt_hbm, in_t)
        for g in range(2):
            for s in range(8):
                rows = 16 * g + 2 * s + duo
                osl = pl.ds(16 * s, 16)
                for j in range(3):
                    gj = plsc.load_gather(in_t, [rows, jvec[j]])
                    out_v[16 + g, j, osl] = gj * 2 + offv[j]
                g3 = plsc.load_gather(in_t, [rows, jvec[3]])
                out_v[16 + g, 3, osl] = g3
        pltpu.sync_copy(out_v.at[pl.ds(0, 18)],
                        out_hbm.at[pl.ds(FULL_CHUNKS * TPC, 18)])


def kernel(voxel_inds):
    xh = voxel_inds[:N_HEAD].reshape(N_IN_TILES, 128, 4).transpose(0, 2, 1)
    xt = voxel_inds[N_HEAD:]
    mesh = plsc.VectorSubcoreMesh(core_axis_name="c", subcore_axis_name="s")
    out3 = pl.kernel(
        _tec_body,
        out_type=jax.ShapeDtypeStruct((N_OUT_TILES, 4, 128), jnp.int32),
        mesh=mesh,
        compiler_params=pltpu.CompilerParams(
            needs_layout_passes=False, use_tc_tiling_on_sc=False),
        scratch_types=[
            pltpu.VMEM((2 * IPC, 4, 128), jnp.int32),
            pltpu.VMEM((N_TAIL, 4), jnp.int32),
            pltpu.VMEM((2 * TPC, 4, 128), jnp.int32),
            pltpu.SemaphoreType.DMA,
            pltpu.SemaphoreType.DMA,
        ],
    )(xh, xt)
    return out3.transpose(0, 2, 1).reshape(-1, 4)


# speedup vs baseline: 22.4655x; 1.1165x over previous
import jax
import jax.numpy as jnp
from jax import lax
from jax.experimental import pallas as pl
from jax.experimental.pallas import tpu as pltpu
from jax.experimental.pallas import tpu_sc as plsc

N_VOX = 500_000
N_HEAD = N_VOX // 128 * 128
N_IN_TILES = N_HEAD // 128
N_TAIL = N_VOX - N_HEAD
N_OUT_TILES = N_VOX * 8 // 128
NW = 32
CH = 1024
IPC = CH // 128
TPC = CH // 16
FULL_CHUNKS = N_HEAD // CH
HEAD_REM_VOX = N_HEAD - FULL_CHUNKS * CH
BASE_ITERS = FULL_CHUNKS // NW
EXTRA_W = FULL_CHUNKS % NW

_OFF_BITS = (210, 180, 232, 0)
_MULS = (2, 2, 2, 1)


def _tec_body(xh_hbm, xt_hbm, out_hbm, in_v, in_t, out_v, sem_in, sem_out):
    nc = 2
    w = lax.axis_index("s") * nc + lax.axis_index("c")

    lane = lax.iota(jnp.int32, 16)
    duo = lane // 8
    corner = lane % 8
    offv = [((b >> corner) & 1).astype(jnp.int32) for b in _OFF_BITS[:3]]
    jvec = [jnp.broadcast_to(jnp.int32(j), (16,)) for j in range(4)]

    nits = jnp.where(w < EXTRA_W, BASE_ITERS + 1, BASE_ITERS)

    def start_in(i, buf):
        cid = w + NW * i
        pltpu.make_async_copy(
            xh_hbm.at[pl.ds(cid * IPC, IPC)],
            in_v.at[pl.ds(buf * IPC, IPC)], sem_in).start()

    def wait_in():
        pltpu.make_async_copy(
            xh_hbm.at[pl.ds(0, IPC)], in_v.at[pl.ds(0, IPC)], sem_in).wait()

    def start_out(i, buf):
        cid = w + NW * i
        pltpu.make_async_copy(
            out_v.at[pl.ds(buf * TPC, TPC)],
            out_hbm.at[pl.ds(cid * TPC, TPC)], sem_out).start()

    def wait_out():
        pltpu.make_async_copy(
            out_v.at[pl.ds(0, TPC)],
            out_hbm.at[pl.ds(0, TPC)], sem_out).wait()

    def compute_chunk(buf):
        ib = buf * IPC
        ob = buf * TPC

        def dbl(bt, carry):
            t = ib + bt
            for j in range(3):
                for v in range(8):
                    sl = pl.ds(16 * v, 16)
                    in_v[t, j, sl] = in_v[t, j, sl] * 2
            return carry
        lax.fori_loop(0, IPC, dbl, 0)

        def blk(bt, carry):
            t = ib + bt
            og0 = ob + 8 * bt
            for gg in range(8):
                for s in range(8):
                    lv = duo + (16 * gg + 2 * s)
                    osl = pl.ds(16 * s, 16)
                    for j in range(3):
                        gj = plsc.load_gather(in_v.at[t, j], [lv])
                        out_v[og0 + gg, j, osl] = gj + offv[j]
                    g3 = plsc.load_gather(in_v.at[t, 3], [lv])
                    out_v[og0 + gg, 3, osl] = g3
            return carry
        lax.fori_loop(0, IPC, blk, 0)

    start_in(0, 0)

    def loop_body(i, carry):
        buf = i & 1
        wait_in()

        @pl.when(i + 1 < nits)
        def _():
            start_in(i + 1, buf ^ 1)

        @pl.when(i >= 2)
        def _():
            wait_out()

        compute_chunk(buf)
        start_out(i, buf)
        return carry
    lax.fori_loop(0, nits, loop_body, 0)

    wait_out()
    wait_out()

    @pl.when(w == EXTRA_W)
    def _():
        pltpu.sync_copy(xh_hbm.at[pl.ds(FULL_CHUNKS * IPC, 2)],
                        in_v.at[pl.ds(0, 2)])
        for bt in range(2):
            for j in range(3):
                for v in range(8):
                    sl = pl.ds(16 * v, 16)
                    in_v[bt, j, sl] = in_v[bt, j, sl] * 2
            for gg in range(8):
                for s in range(8):
                    lv = duo + (16 * gg + 2 * s)
                    osl = pl.ds(16 * s, 16)
                    for j in range(3):
                        gj = plsc.load_gather(in_v.at[bt, j], [lv])
                        out_v[8 * bt + gg, j, osl] = gj + offv[j]
                    g3 = plsc.load_gather(in_v.at[bt, 3], [lv])
                    out_v[8 * bt + gg, 3, osl] = g3
        pltpu.sync_copy(xt_hbm, in_t)
        for g in range(2):
            for s in range(8):
                rows = 16 * g + 2 * s + duo
                osl = pl.ds(16 * s, 16)
                for j in range(3):
                    gj = plsc.load_gather(in_t, [rows, jvec[j]])
                    out_v[16 + g, j, osl] = gj * 2 + offv[j]
                g3 = plsc.load_gather(in_t, [rows, jvec[3]])
                out_v[16 + g, 3, osl] = g3
        pltpu.sync_copy(out_v.at[pl.ds(0, 18)],
                        out_hbm.at[pl.ds(FULL_CHUNKS * TPC, 18)])


def kernel(voxel_inds):
    xh = voxel_inds[:N_HEAD].reshape(N_IN_TILES, 128, 4).transpose(0, 2, 1)
    xt = voxel_inds[N_HEAD:]
    mesh = plsc.VectorSubcoreMesh(core_axis_name="c", subcore_axis_name="s")
    out3 = pl.kernel(
        _tec_body,
        out_type=jax.ShapeDtypeStruct((N_OUT_TILES, 4, 128), jnp.int32),
        mesh=mesh,
        compiler_params=pltpu.CompilerParams(
            needs_layout_passes=False, use_tc_tiling_on_sc=False),
        scratch_types=[
            pltpu.VMEM((2 * IPC, 4, 128), jnp.int32),
            pltpu.VMEM((N_TAIL, 4), jnp.int32),
            pltpu.VMEM((2 * TPC, 4, 128), jnp.int32),
            pltpu.SemaphoreType.DMA,
            pltpu.SemaphoreType.DMA,
        ],
    )(xh, xt)
    return out3.transpose(0, 2, 1).reshape(-1, 4)


# speedup vs baseline: 54.9868x; 2.4476x over previous
import jax
import jax.numpy as jnp
from jax import lax
from jax.experimental import pallas as pl
from jax.experimental.pallas import tpu as pltpu
from jax.experimental.pallas import tpu_sc as plsc

N_VOX = 500_000
N_HEAD = N_VOX // 128 * 128
N_IN_TILES = N_HEAD // 128
N_TAIL = N_VOX - N_HEAD
N_OUT_TILES = N_VOX * 8 // 128
NW = 32
CH = 1024
IPC = CH // 128
TPC = CH // 16
FULL_CHUNKS = N_HEAD // CH
HEAD_REM_VOX = N_HEAD - FULL_CHUNKS * CH
BASE_ITERS = FULL_CHUNKS // NW
EXTRA_W = FULL_CHUNKS % NW

_OFF_BITS = (210, 180, 232, 0)
_MULS = (2, 2, 2, 1)


def _tec_body(xh_hbm, xt_hbm, out_hbm, in_v, in_t, out_v, sem_in, sem_out):
    nc = 2
    w = lax.axis_index("s") * nc + lax.axis_index("c")

    lane = lax.iota(jnp.int32, 16)
    duo = lane // 8
    corner = lane % 8
    offv = [((b >> corner) & 1).astype(jnp.int32) for b in _OFF_BITS[:3]]
    jvec = [jnp.broadcast_to(jnp.int32(j), (16,)) for j in range(4)]

    nits = jnp.where(w < EXTRA_W, BASE_ITERS + 1, BASE_ITERS)

    def start_in(i, buf):
        cid = w + NW * i
        pltpu.make_async_copy(
            xh_hbm.at[pl.ds(cid * IPC, IPC)],
            in_v.at[pl.ds(buf * IPC, IPC)], sem_in).start()

    def wait_in():
        pltpu.make_async_copy(
            xh_hbm.at[pl.ds(0, IPC)], in_v.at[pl.ds(0, IPC)], sem_in).wait()

    def start_out(i, buf):
        cid = w + NW * i
        pltpu.make_async_copy(
            out_v.at[pl.ds(buf * TPC, TPC)],
            out_hbm.at[pl.ds(cid * TPC, TPC)], sem_out).start()

    def wait_out():
        pltpu.make_async_copy(
            out_v.at[pl.ds(0, TPC)],
            out_hbm.at[pl.ds(0, TPC)], sem_out).wait()

    idxs = [lane * 8 + s for s in range(8)]

    def compute_chunk(buf):
        ib = buf * IPC
        ob = buf * TPC

        def blk(bt, carry):
            t = ib + bt
            og0 = ob + 8 * bt
            for gg in range(8):
                og = og0 + gg
                isl = pl.ds(16 * gg, 16)
                for j in range(4):
                    src = in_v[t, j, isl]
                    if j < 3:
                        src = src + src
                    for s in range(8):
                        data = src + 1 if (_OFF_BITS[j] >> s) & 1 else src
                        plsc.store_scatter(out_v.at[og, j], [idxs[s]], data)
            return carry
        lax.fori_loop(0, IPC, blk, 0)

    start_in(0, 0)

    def loop_body(i, carry):
        buf = i & 1
        wait_in()

        @pl.when(i + 1 < nits)
        def _():
            start_in(i + 1, buf ^ 1)

        @pl.when(i >= 2)
        def _():
            wait_out()

        compute_chunk(buf)
        start_out(i, buf)
        return carry
    lax.fori_loop(0, nits, loop_body, 0)

    wait_out()
    wait_out()

    @pl.when(w == EXTRA_W)
    def _():
        pltpu.sync_copy(xh_hbm.at[pl.ds(FULL_CHUNKS * IPC, 2)],
                        in_v.at[pl.ds(0, 2)])
        for bt in range(2):
            for j in range(3):
                for v in range(8):
                    sl = pl.ds(16 * v, 16)
                    in_v[bt, j, sl] = in_v[bt, j, sl] * 2
            for gg in range(8):
                for s in range(8):
                    lv = duo + (16 * gg + 2 * s)
                    osl = pl.ds(16 * s, 16)
                    for j in range(3):
                        gj = plsc.load_gather(in_v.at[bt, j], [lv])
                        out_v[8 * bt + gg, j, osl] = gj + offv[j]
                    g3 = plsc.load_gather(in_v.at[bt, 3], [lv])
                    out_v[8 * bt + gg, 3, osl] = g3
        pltpu.sync_copy(xt_hbm, in_t)
        for g in range(2):
            for s in range(8):
                rows = 16 * g + 2 * s + duo
                osl = pl.ds(16 * s, 16)
                for j in range(3):
                    gj = plsc.load_gather(in_t, [rows, jvec[j]])
                    out_v[16 + g, j, osl] = gj * 2 + offv[j]
                g3 = plsc.load_gather(in_t, [rows, jvec[3]])
                out_v[16 + g, 3, osl] = g3
        pltpu.sync_copy(out_v.at[pl.ds(0, 18)],
                        out_hbm.at[pl.ds(FULL_CHUNKS * TPC, 18)])


def kernel(voxel_inds):
    xh = voxel_inds[:N_HEAD].reshape(N_IN_TILES, 128, 4).transpose(0, 2, 1)
    xt = voxel_inds[N_HEAD:]
    mesh = plsc.VectorSubcoreMesh(core_axis_name="c", subcore_axis_name="s")
    out3 = pl.kernel(
        _tec_body,
        out_type=jax.ShapeDtypeStruct((N_OUT_TILES, 4, 128), jnp.int32),
        mesh=mesh,
        compiler_params=pltpu.CompilerParams(
            needs_layout_passes=False, use_tc_tiling_on_sc=False),
        scratch_types=[
            pltpu.VMEM((2 * IPC, 4, 128), jnp.int32),
            pltpu.VMEM((N_TAIL, 4), jnp.int32),
            pltpu.VMEM((2 * TPC, 4, 128), jnp.int32),
            pltpu.SemaphoreType.DMA,
            pltpu.SemaphoreType.DMA,
        ],
    )(xh, xt)
    return out3.transpose(0, 2, 1).reshape(-1, 4)


# speedup vs baseline: 55.0387x; 1.0009x over previous
import jax
import jax.numpy as jnp
from jax import lax
from jax.experimental import pallas as pl
from jax.experimental.pallas import tpu as pltpu
from jax.experimental.pallas import tpu_sc as plsc

N_VOX = 500_000
N_HEAD = N_VOX // 128 * 128
N_IN_TILES = N_HEAD // 128
N_TAIL = N_VOX - N_HEAD
N_OUT_TILES = N_VOX * 8 // 128
NW = 32
CH = 1024
IPC = CH // 128
TPC = CH // 16
FULL_CHUNKS = N_HEAD // CH
HEAD_REM_VOX = N_HEAD - FULL_CHUNKS * CH
BASE_ITERS = FULL_CHUNKS // NW
EXTRA_W = FULL_CHUNKS % NW

_OFF_BITS = (210, 180, 232, 0)
_MULS = (2, 2, 2, 1)


def _tec_body(xh_hbm, xt_hbm, out_hbm, in_v, in_t, out_v, sem_in, sem_out):
    nc = 2
    w = lax.axis_index("s") * nc + lax.axis_index("c")

    lane = lax.iota(jnp.int32, 16)
    duo = lane // 8
    corner = lane % 8
    offv = [((b >> corner) & 1).astype(jnp.int32) for b in _OFF_BITS[:3]]
    jvec = [jnp.broadcast_to(jnp.int32(j), (16,)) for j in range(4)]

    nits = jnp.where(w < EXTRA_W, BASE_ITERS + 1, BASE_ITERS)

    def start_in(i, buf):
        cid = w + NW * i
        pltpu.make_async_copy(
            xh_hbm.at[pl.ds(cid * IPC, IPC)],
            in_v.at[pl.ds(buf * IPC, IPC)], sem_in).start()

    def wait_in():
        pltpu.make_async_copy(
            xh_hbm.at[pl.ds(0, IPC)], in_v.at[pl.ds(0, IPC)], sem_in).wait()

    def start_out(i, buf):
        cid = w + NW * i
        pltpu.make_async_copy(
            out_v.at[pl.ds(buf * TPC, TPC)],
            out_hbm.at[pl.ds(cid * TPC, TPC)], sem_out).start()

    def wait_out():
        pltpu.make_async_copy(
            out_v.at[pl.ds(0, TPC)],
            out_hbm.at[pl.ds(0, TPC)], sem_out).wait()

    idxs = [lane * 8 + s for s in range(8)]

    def compute_chunk(buf):
        ib = buf * IPC
        ob = buf * TPC

        def blk(bt, carry):
            t = ib + bt
            og0 = ob + 8 * bt
            for gg in range(8):
                og = og0 + gg
                isl = pl.ds(16 * gg, 16)
                for j in range(4):
                    src = in_v[t, j, isl]
                    if j < 3:
                        src = src + src
                        srcp1 = src + 1
                    else:
                        srcp1 = src
                    for s in range(8):
                        data = srcp1 if (_OFF_BITS[j] >> s) & 1 else src
                        plsc.store_scatter(out_v.at[og, j], [idxs[s]], data)
            return carry
        lax.fori_loop(0, IPC, blk, 0)

    start_in(0, 0)

    def loop_body(i, carry):
        buf = i & 1
        wait_in()

        @pl.when(i + 1 < nits)
        def _():
            start_in(i + 1, buf ^ 1)

        @pl.when(i >= 2)
        def _():
            wait_out()

        compute_chunk(buf)
        start_out(i, buf)
        return carry
    lax.fori_loop(0, nits, loop_body, 0)

    wait_out()
    wait_out()

    @pl.when(w == EXTRA_W)
    def _():
        pltpu.sync_copy(xh_hbm.at[pl.ds(FULL_CHUNKS * IPC, 2)],
                        in_v.at[pl.ds(0, 2)])
        for bt in range(2):
            for j in range(3):
                for v in range(8):
                    sl = pl.ds(16 * v, 16)
                    in_v[bt, j, sl] = in_v[bt, j, sl] * 2
            for gg in range(8):
                for s in range(8):
                    lv = duo + (16 * gg + 2 * s)
                    osl = pl.ds(16 * s, 16)
                    for j in range(3):
                        gj = plsc.load_gather(in_v.at[bt, j], [lv])
                        out_v[8 * bt + gg, j, osl] = gj + offv[j]
                    g3 = plsc.load_gather(in_v.at[bt, 3], [lv])
                    out_v[8 * bt + gg, 3, osl] = g3
        pltpu.sync_copy(xt_hbm, in_t)
        for g in range(2):
            for s in range(8):
                rows = 16 * g + 2 * s + duo
                osl = pl.ds(16 * s, 16)
                for j in range(3):
                    gj = plsc.load_gather(in_t, [rows, jvec[j]])
                    out_v[16 + g, j, osl] = gj * 2 + offv[j]
                g3 = plsc.load_gather(in_t, [rows, jvec[3]])
                out_v[16 + g, 3, osl] = g3
        pltpu.sync_copy(out_v.at[pl.ds(0, 18)],
                        out_hbm.at[pl.ds(FULL_CHUNKS * TPC, 18)])


def kernel(voxel_inds):
    xh = voxel_inds[:N_HEAD].reshape(N_IN_TILES, 128, 4).transpose(0, 2, 1)
    xt = voxel_inds[N_HEAD:]
    mesh = plsc.VectorSubcoreMesh(core_axis_name="c", subcore_axis_name="s")
    out3 = pl.kernel(
        _tec_body,
        out_type=jax.ShapeDtypeStruct((N_OUT_TILES, 4, 128), jnp.int32),
        mesh=mesh,
        compiler_params=pltpu.CompilerParams(
            needs_layout_passes=False, use_tc_tiling_on_sc=False),
        scratch_types=[
            pltpu.VMEM((2 * IPC, 4, 128), jnp.int32),
            pltpu.VMEM((N_TAIL, 4), jnp.int32),
            pltpu.VMEM((2 * TPC, 4, 128), jnp.int32),
            pltpu.SemaphoreType.DMA,
            pltpu.SemaphoreType.DMA,
        ],
    )(xh, xt)
    return out3.transpose(0, 2, 1).reshape(-1, 4)
